# baseline (speedup 1.0000x reference)
import functools
import math

import jax
import jax.numpy as jnp
import numpy as np
from jax import lax
from jax.experimental import pallas as pl
from jax.experimental.pallas import tpu as pltpu
from jax.experimental.pallas import tpu_sc as plsc

_VOCAB = 1000000
_HIDDEN = 64
_BATCH = 4096
_SEQ = 200
_EPS = 1e-6

_NW = 32
_ROWS_PER_W = _BATCH // _NW
_CHUNKS = ((0, 104), (104, 96))
_NCH = _HIDDEN // 16


def _pos_enc(seq_len, hidden_dim):
    position = np.arange(seq_len, dtype=np.float32)[:, None]
    div_term = np.exp(
        np.arange(0, hidden_dim, 2, dtype=np.float32)
        * (-math.log(10000.0) / hidden_dim)
    )
    pe = np.zeros((seq_len, hidden_dim), dtype=np.float32)
    pe[:, 0::2] = np.sin(position * div_term)
    pe[:, 1::2] = np.cos(position * div_term)
    return jnp.asarray(pe)


@functools.partial(
    pl.kernel,
    out_type=jax.ShapeDtypeStruct((_BATCH, _SEQ, _HIDDEN), jnp.float32),
    mesh=plsc.VectorSubcoreMesh(core_axis_name="c", subcore_axis_name="s"),
    compiler_params=pltpu.CompilerParams(use_tc_tiling_on_sc=False),
    scratch_types=[
        pltpu.VMEM((_SEQ, _HIDDEN), jnp.float32),
        pltpu.VMEM((_HIDDEN,), jnp.float32),
        pltpu.VMEM((_HIDDEN,), jnp.float32),
        pltpu.VMEM((_SEQ,), jnp.int32),
        pltpu.VMEM((_SEQ, _HIDDEN), jnp.float32),
        pltpu.VMEM((_SEQ, _HIDDEN), jnp.float32),
        pltpu.SemaphoreType.DMA,
    ],
)
def _emb_ln(x_hbm, tab_hbm, pe_hbm, a_hbm, b_hbm, out_hbm,
            pe_v, a_v, b_v, idx_v, rows_v, out_v, sem):
    wid = lax.axis_index("s") * 2 + lax.axis_index("c")
    pltpu.sync_copy(pe_hbm, pe_v)
    pltpu.sync_copy(a_hbm, a_v)
    pltpu.sync_copy(b_hbm, b_v)
    a_c = [a_v[pl.ds(16 * c, 16)] for c in range(_NCH)]
    ab_c = [a_c[c] * b_v[pl.ds(16 * c, 16)] for c in range(_NCH)]

    iota = lax.iota(jnp.int32, 16)
    perms = [jnp.bitwise_xor(iota, np.int32(k)) for k in (1, 2, 4, 8)]

    def lane_sum(v):
        for p in perms:
            v = v + v.at[p].get(mode="promise_in_bounds")
        return v

    row0 = wid * _ROWS_PER_W

    def token(s2, carry):
        y = [rows_v[s2, pl.ds(16 * c, 16)] + pe_v[s2, pl.ds(16 * c, 16)]
             for c in range(_NCH)]
        sv = (y[0] + y[1]) + (y[2] + y[3])
        qv = (y[0] * y[0] + y[1] * y[1]) + (y[2] * y[2] + y[3] * y[3])
        ssum = lane_sum(sv)
        ssq = lane_sum(qv)
        mean = ssum * np.float32(1.0 / 64.0)
        var = (ssq - ssum * mean) * np.float32(1.0 / 63.0)
        var = jnp.maximum(var, np.float32(0.0))
        ii = lax.bitcast_convert_type(var, jnp.int32)
        ii = np.int32(0x5F3759DF) - lax.shift_right_arithmetic(ii, 1)
        r = lax.bitcast_convert_type(ii, jnp.float32)
        for _ in range(3):
            r = r * (np.float32(1.5) - np.float32(0.5) * var * r * r)
        sigma = var * r + np.float32(_EPS)
        inv = np.float32(1.0) / sigma
        for c in range(_NCH):
            out_v[s2, pl.ds(16 * c, 16)] = \
                (y[c] - mean) * inv * a_c[c] + ab_c[c]
        return carry

    def process_row(i, carry):
        b = row0 + i
        pltpu.sync_copy(x_hbm.at[b], idx_v)
        for off, n in _CHUNKS:
            pltpu.async_copy(
                tab_hbm.at[idx_v.at[pl.ds(off, n)]],
                rows_v.at[pl.ds(off, n)],
                sem,
            ).wait()
        lax.fori_loop(0, _SEQ, token, 0)
        pltpu.sync_copy(out_v, out_hbm.at[b])
        return carry

    lax.fori_loop(0, _ROWS_PER_W, process_row, 0)


def kernel(x, emb_table, alpha, beta):
    pe = _pos_enc(_SEQ, _HIDDEN)
    return _emb_ln(x, emb_table, pe, alpha, beta)
